# Initial kernel scaffold; baseline (speedup 1.0000x reference)
import functools

import jax
import jax.numpy as jnp
import numpy as np
from jax import lax
from jax.experimental import pallas as pl
from jax.experimental.pallas import tpu as pltpu
from jax.experimental.pallas import tpu_sc as plsc

_PRE_N = 100000
_POST_N = 100000
_N_EDGES = 6400000
_DT = 0.1
_TAU_AMPA = 2.0
_E_AMPA = 0.0
_G_BAR = 1.0
_DECAY = float(np.exp(-_DT / _TAU_AMPA))

_NC = 2
_NS = 16
_NW = _NC * _NS
_L = 16

_CH = 2048
_NCHUNK = _N_EDGES // _CH
_CPW = -(-_NCHUNK // _NW)

_ZCH = 2000
_NZ = _POST_N // _ZCH

_SCH = 2000
_NS_CH = _PRE_N // _SCH


def _sc_partial_g(pre_spikes, s, col2, val2, row2):
    mesh = plsc.VectorSubcoreMesh(core_axis_name="c", subcore_axis_name="s")

    @functools.partial(
        pl.kernel,
        mesh=mesh,
        out_type=jax.ShapeDtypeStruct((_NC, _POST_N), jnp.float32),
        scratch_types=[
            pltpu.VMEM((_PRE_N,), jnp.float32),
            pltpu.VMEM((_CH,), jnp.int32),
            pltpu.VMEM((_CH,), jnp.float32),
            pltpu.VMEM((_CH,), jnp.int32),
            pltpu.VMEM((_CH,), jnp.float32),
            pltpu.VMEM((_CH,), jnp.float32),
            pltpu.VMEM_SHARED((_POST_N,), jnp.float32),
        ],
    )
    def kern(pre_hbm, s_hbm, col_hbm, val_hbm, row_hbm, out_hbm,
             s_tile, colb, valb, rowb, conb, preb, g_sh):
        cid = lax.axis_index("c")
        sid = lax.axis_index("s")
        wid = cid * _NS + sid

        pltpu.sync_copy(s_hbm, s_tile)

        def s_chunk(c, _):
            pltpu.sync_copy(pre_hbm.at[pl.ds(c * _SCH, _SCH)],
                            preb.at[pl.ds(0, _SCH)])

            def s_vec(i, _):
                off = c * _SCH + i * _L
                s_tile[pl.ds(off, _L)] = (
                    s_tile[pl.ds(off, _L)] * _DECAY + preb[pl.ds(i * _L, _L)])
                return 0

            lax.fori_loop(0, _SCH // _L, s_vec, 0)
            return 0

        lax.fori_loop(0, _NS_CH, s_chunk, 0)

        def zero_vec(i, _):
            conb[pl.ds(i * _L, _L)] = jnp.zeros((_L,), jnp.float32)
            return 0

        lax.fori_loop(0, _CH // _L, zero_vec, 0)

        for k in range(3):
            pltpu.sync_copy(conb.at[pl.ds(0, _ZCH)],
                            g_sh.at[pl.ds((sid + _NS * k) * _ZCH, _ZCH)])

        @pl.when(sid < _NZ - 3 * _NS)
        def _():
            pltpu.sync_copy(conb.at[pl.ds(0, _ZCH)],
                            g_sh.at[pl.ds((sid + _NS * 3) * _ZCH, _ZCH)])

        plsc.subcore_barrier()

        def edge_chunk(k, _):
            c = wid + _NW * k

            @pl.when(c < _NCHUNK)
            def _():
                pltpu.sync_copy(col_hbm.at[c], colb)
                pltpu.sync_copy(val_hbm.at[c], valb)
                pltpu.sync_copy(row_hbm.at[c], rowb)

                def gmul(i, _):
                    sl = pl.ds(i * _L, _L)
                    idx = colb[sl]
                    sv = plsc.load_gather(s_tile, [idx])
                    conb[sl] = valb[sl] * sv
                    return 0

                lax.fori_loop(0, _CH // _L, gmul, 0)
                pltpu.sync_copy(conb, g_sh.at[rowb], add=True)

            return 0

        lax.fori_loop(0, _CPW, edge_chunk, 0)
        plsc.subcore_barrier()

        for k in range(3):
            z = sid + _NS * k
            pltpu.sync_copy(g_sh.at[pl.ds(z * _ZCH, _ZCH)],
                            conb.at[pl.ds(0, _ZCH)])
            pltpu.sync_copy(conb.at[pl.ds(0, _ZCH)],
                            out_hbm.at[cid, pl.ds(z * _ZCH, _ZCH)])

        @pl.when(sid < _NZ - 3 * _NS)
        def _():
            z = sid + _NS * 3
            pltpu.sync_copy(g_sh.at[pl.ds(z * _ZCH, _ZCH)],
                            conb.at[pl.ds(0, _ZCH)])
            pltpu.sync_copy(conb.at[pl.ds(0, _ZCH)],
                            out_hbm.at[cid, pl.ds(z * _ZCH, _ZCH)])

    return kern(pre_spikes, s, col2, val2, row2)


_PAD_N = 100352
_ROWS = _PAD_N // 128


def _tc_finish(partial, post_v):
    a = jnp.pad(partial, ((0, 0), (0, _PAD_N - _POST_N))).reshape(
        _NC, _ROWS, 128)
    b = jnp.pad(post_v, (0, _PAD_N - _POST_N)).reshape(_ROWS, 128)

    def fin(a_ref, b_ref, o_ref):
        g = a_ref[0] + a_ref[1]
        o_ref[...] = (_G_BAR * g) * (_E_AMPA - b_ref[...])

    out = pl.pallas_call(
        fin,
        out_shape=jax.ShapeDtypeStruct((_ROWS, 128), jnp.float32),
    )(a, b)
    return out.reshape(-1)[:_POST_N]


def kernel(pre_spikes, post_v, _row, _col, _val, s):
    col2 = _col.astype(jnp.int32).reshape(_NCHUNK, _CH)
    row2 = _row.astype(jnp.int32).reshape(_NCHUNK, _CH)
    val2 = _val.reshape(_NCHUNK, _CH)
    partial = _sc_partial_g(pre_spikes, s, col2, val2, row2)
    return _tc_finish(partial, post_v)


# speedup vs baseline: 131.6048x; 131.6048x over previous
import functools

import jax
import jax.numpy as jnp
import numpy as np
from jax import lax
from jax.experimental import pallas as pl
from jax.experimental.pallas import tpu as pltpu
from jax.experimental.pallas import tpu_sc as plsc

_PRE_N = 100000
_POST_N = 100000
_N_EDGES = 6400000
_DT = 0.1
_TAU_AMPA = 2.0
_E_AMPA = 0.0
_G_BAR = 1.0
_DECAY = float(np.exp(-_DT / _TAU_AMPA))

_NC = 2
_NS = 16
_NW = _NC * _NS
_L = 16

_CH = 2048
_NCHUNK = _N_EDGES // _CH
_CPW = -(-_NCHUNK // _NW)

_ZCH = 2000
_NZ = _POST_N // _ZCH

_SCH = 2000
_NS_CH = _PRE_N // _SCH


def _sc_partial_g(pre_spikes, s, col2, val2, row2):
    mesh = plsc.VectorSubcoreMesh(core_axis_name="c", subcore_axis_name="s")

    @functools.partial(
        pl.kernel,
        mesh=mesh,
        out_type=jax.ShapeDtypeStruct((_NC * _POST_N,), jnp.float32),
        compiler_params=pltpu.CompilerParams(needs_layout_passes=False),
        scratch_types=[
            pltpu.VMEM((_PRE_N,), jnp.float32),
            pltpu.VMEM((_CH,), jnp.int32),
            pltpu.VMEM((_CH,), jnp.float32),
            pltpu.VMEM((_CH,), jnp.int32),
            pltpu.VMEM((_CH,), jnp.float32),
            pltpu.VMEM((_CH,), jnp.float32),
            pltpu.VMEM_SHARED((_POST_N,), jnp.float32),
        ],
    )
    def kern(pre_hbm, s_hbm, col_hbm, val_hbm, row_hbm, out_hbm,
             s_tile, colb, valb, rowb, conb, preb, g_sh):
        cid = lax.axis_index("c")
        sid = lax.axis_index("s")
        wid = cid * _NS + sid

        pltpu.sync_copy(s_hbm, s_tile)

        def s_chunk(c, _):
            pltpu.sync_copy(pre_hbm.at[pl.ds(c * _SCH, _SCH)],
                            preb.at[pl.ds(0, _SCH)])

            def s_vec(i, _):
                off = c * _SCH + i * _L
                s_tile[pl.ds(off, _L)] = (
                    s_tile[pl.ds(off, _L)] * _DECAY + preb[pl.ds(i * _L, _L)])
                return 0

            lax.fori_loop(0, _SCH // _L, s_vec, 0)
            return 0

        lax.fori_loop(0, _NS_CH, s_chunk, 0)

        def zero_vec(i, _):
            conb[pl.ds(i * _L, _L)] = jnp.zeros((_L,), jnp.float32)
            return 0

        lax.fori_loop(0, _CH // _L, zero_vec, 0)

        for k in range(3):
            pltpu.sync_copy(conb.at[pl.ds(0, _ZCH)],
                            g_sh.at[pl.ds((sid + _NS * k) * _ZCH, _ZCH)])

        @pl.when(sid < _NZ - 3 * _NS)
        def _():
            pltpu.sync_copy(conb.at[pl.ds(0, _ZCH)],
                            g_sh.at[pl.ds((sid + _NS * 3) * _ZCH, _ZCH)])

        plsc.subcore_barrier()

        def edge_chunk(k, _):
            c = wid + _NW * k

            @pl.when(c < _NCHUNK)
            def _():
                pltpu.sync_copy(col_hbm.at[c], colb)
                pltpu.sync_copy(val_hbm.at[c], valb)
                pltpu.sync_copy(row_hbm.at[c], rowb)

                def gmul(i, _):
                    sl = pl.ds(i * _L, _L)
                    idx = colb[sl]
                    sv = plsc.load_gather(s_tile, [idx])
                    conb[sl] = valb[sl] * sv
                    return 0

                lax.fori_loop(0, _CH // _L, gmul, 0)
                pltpu.sync_copy(conb, g_sh.at[rowb], add=True)

            return 0

        lax.fori_loop(0, _CPW, edge_chunk, 0)
        plsc.subcore_barrier()

        for k in range(3):
            z = sid + _NS * k
            pltpu.sync_copy(g_sh.at[pl.ds(z * _ZCH, _ZCH)],
                            conb.at[pl.ds(0, _ZCH)])
            pltpu.sync_copy(conb.at[pl.ds(0, _ZCH)],
                            out_hbm.at[pl.ds(cid * _POST_N + z * _ZCH, _ZCH)])

        @pl.when(sid < _NZ - 3 * _NS)
        def _():
            z = sid + _NS * 3
            pltpu.sync_copy(g_sh.at[pl.ds(z * _ZCH, _ZCH)],
                            conb.at[pl.ds(0, _ZCH)])
            pltpu.sync_copy(conb.at[pl.ds(0, _ZCH)],
                            out_hbm.at[pl.ds(cid * _POST_N + z * _ZCH, _ZCH)])

    return kern(pre_spikes, s, col2, val2, row2)


_PAD_N = 100352
_ROWS = _PAD_N // 128


def _tc_finish(partial, post_v):
    a = jnp.pad(partial, ((0, 0), (0, _PAD_N - _POST_N))).reshape(
        _NC, _ROWS, 128)
    b = jnp.pad(post_v, (0, _PAD_N - _POST_N)).reshape(_ROWS, 128)

    def fin(a_ref, b_ref, o_ref):
        g = a_ref[0] + a_ref[1]
        o_ref[...] = (_G_BAR * g) * (_E_AMPA - b_ref[...])

    out = pl.pallas_call(
        fin,
        out_shape=jax.ShapeDtypeStruct((_ROWS, 128), jnp.float32),
    )(a, b)
    return out.reshape(-1)[:_POST_N]


def kernel(pre_spikes, post_v, _row, _col, _val, s):
    col2 = _col.astype(jnp.int32).reshape(_NCHUNK, _CH)
    row2 = _row.astype(jnp.int32).reshape(_NCHUNK, _CH)
    val2 = _val.reshape(_NCHUNK, _CH)
    partial = _sc_partial_g(pre_spikes, s, col2, val2, row2)
    return _tc_finish(partial.reshape(_NC, _POST_N), post_v)


# speedup vs baseline: 342.9229x; 2.6057x over previous
import functools

import jax
import jax.numpy as jnp
import numpy as np
from jax import lax
from jax.experimental import pallas as pl
from jax.experimental.pallas import tpu as pltpu
from jax.experimental.pallas import tpu_sc as plsc

_PRE_N = 100000
_POST_N = 100000
_N_EDGES = 6400000
_DT = 0.1
_TAU_AMPA = 2.0
_E_AMPA = 0.0
_G_BAR = 1.0
_DECAY = float(np.exp(-_DT / _TAU_AMPA))

_NC = 2
_NS = 16
_NW = _NC * _NS
_L = 16

_CH = 2048
_NCHUNK = _N_EDGES // _CH
_NJ = -(-(-(-_NCHUNK // _NW) + 1) // 3)

_ZCH = 2000
_NZ = _POST_N // _ZCH

_PAD_N = 100352
_ROWS = _PAD_N // 128


def _sc_partial_g(snew, col2, val2, row2):
    mesh = plsc.VectorSubcoreMesh(core_axis_name="c", subcore_axis_name="s")

    @functools.partial(
        pl.kernel,
        mesh=mesh,
        out_type=jax.ShapeDtypeStruct((_NC * _POST_N,), jnp.float32),
        compiler_params=pltpu.CompilerParams(needs_layout_passes=False),
        scratch_types=[
            pltpu.VMEM((_PRE_N,), jnp.float32),
            pltpu.VMEM((_CH,), jnp.int32),
            pltpu.VMEM((_CH,), jnp.int32),
            pltpu.VMEM((_CH,), jnp.int32),
            pltpu.VMEM((_CH,), jnp.float32),
            pltpu.VMEM((_CH,), jnp.float32),
            pltpu.VMEM((_CH,), jnp.float32),
            pltpu.VMEM((_CH,), jnp.int32),
            pltpu.VMEM((_CH,), jnp.int32),
            pltpu.VMEM((_CH,), jnp.int32),
            pltpu.VMEM((_CH,), jnp.float32),
            pltpu.VMEM((_CH,), jnp.float32),
            pltpu.VMEM((_CH,), jnp.float32),
            pltpu.VMEM_SHARED((_POST_N,), jnp.float32),
            pltpu.SemaphoreType.DMA,
            pltpu.SemaphoreType.DMA,
            pltpu.SemaphoreType.DMA,
            pltpu.SemaphoreType.DMA,
            pltpu.SemaphoreType.DMA,
            pltpu.SemaphoreType.DMA,
            pltpu.SemaphoreType.DMA,
        ],
    )
    def kern(snew_hbm, col_hbm, val_hbm, row_hbm, out_hbm,
             s_tile, col0, col1, col2, val0, val1, val2,
             row0, row1, row2, con0, con1, con2, g_sh,
             lsem, dsem0, dsem1, dsem2, ssem0, ssem1, ssem2):
        cid = lax.axis_index("c")
        sid = lax.axis_index("s")
        wid = cid * _NS + sid
        colb = (col0, col1, col2)
        valb = (val0, val1, val2)
        rowb = (row0, row1, row2)
        conb = (con0, con1, con2)
        dsems = (dsem0, dsem1, dsem2)
        ssems = (ssem0, ssem1, ssem2)

        pltpu.async_copy(snew_hbm.at[pl.ds(0, _PRE_N)], s_tile, lsem)

        def zero_vec(i, _):
            con0[pl.ds(i * _L, _L)] = jnp.zeros((_L,), jnp.float32)
            return 0

        lax.fori_loop(0, _ZCH // _L, zero_vec, 0)

        for k in range(3):
            pltpu.sync_copy(con0.at[pl.ds(0, _ZCH)],
                            g_sh.at[pl.ds((sid + _NS * k) * _ZCH, _ZCH)])

        @pl.when(sid < _NZ - 3 * _NS)
        def _():
            pltpu.sync_copy(con0.at[pl.ds(0, _ZCH)],
                            g_sh.at[pl.ds((sid + _NS * 3) * _ZCH, _ZCH)])

        pltpu.make_async_copy(snew_hbm.at[pl.ds(0, _PRE_N)], s_tile,
                              lsem).wait()
        plsc.subcore_barrier()

        def dma_start(slot, c):
            pltpu.async_copy(col_hbm.at[c], colb[slot], dsems[slot])
            pltpu.async_copy(val_hbm.at[c], valb[slot], dsems[slot])
            pltpu.async_copy(row_hbm.at[c], rowb[slot], dsems[slot])

        def dma_wait(slot, c):
            pltpu.make_async_copy(col_hbm.at[c], colb[slot],
                                  dsems[slot]).wait()
            pltpu.make_async_copy(val_hbm.at[c], valb[slot],
                                  dsems[slot]).wait()
            pltpu.make_async_copy(row_hbm.at[c], rowb[slot],
                                  dsems[slot]).wait()

        def scat_start(slot):
            pltpu.async_copy(conb[slot], g_sh.at[rowb[slot]],
                             ssems[slot], add=True)

        def scat_wait(slot):
            pltpu.make_async_copy(conb[slot], g_sh.at[rowb[slot]],
                                  ssems[slot]).wait()

        dma_start(0, wid)

        def step(j, b):
            k = 3 * j + b
            c = wid + _NW * k

            @pl.when(c < _NCHUNK)
            def _():
                nxt = (b + 1) % 3

                @pl.when(c + _NW < _NCHUNK)
                def _():
                    if b == 2:
                        scat_wait(nxt)
                    else:
                        @pl.when(j >= 1)
                        def _():
                            scat_wait(nxt)

                    dma_start(nxt, c + _NW)

                dma_wait(b, c)

                def gmul(i, _):
                    sl = pl.ds(i * _L, _L)
                    idx = colb[b][sl]
                    sv = plsc.load_gather(s_tile, [idx])
                    conb[b][sl] = valb[b][sl] * sv
                    return 0

                lax.fori_loop(0, _CH // _L, gmul, 0)
                scat_start(b)

        def body(j, _):
            step(j, 0)
            step(j, 1)
            step(j, 2)
            return 0

        lax.fori_loop(0, _NJ, body, 0)

        for b in range(3):
            scat_wait(b)
        plsc.subcore_barrier()

        def readout(z):
            pltpu.sync_copy(g_sh.at[pl.ds(z * _ZCH, _ZCH)],
                            con0.at[pl.ds(0, _ZCH)])
            pltpu.sync_copy(con0.at[pl.ds(0, _ZCH)],
                            out_hbm.at[pl.ds(cid * _POST_N + z * _ZCH, _ZCH)])

        for k in range(3):
            readout(sid + _NS * k)

        @pl.when(sid < _NZ - 3 * _NS)
        def _():
            readout(sid + _NS * 3)

    return kern(snew, col2, val2, row2)


def _tc_snew(pre_spikes, s):
    a = jnp.pad(s, (0, _PAD_N - _PRE_N)).reshape(_ROWS, 128)
    b = jnp.pad(pre_spikes, (0, _PAD_N - _PRE_N)).reshape(_ROWS, 128)

    def upd(s_ref, p_ref, o_ref):
        o_ref[...] = s_ref[...] * _DECAY + p_ref[...]

    out = pl.pallas_call(
        upd,
        out_shape=jax.ShapeDtypeStruct((_ROWS, 128), jnp.float32),
    )(a, b)
    return out.reshape(-1)


def _tc_finish(partial, post_v):
    a = jnp.pad(partial, ((0, 0), (0, _PAD_N - _POST_N))).reshape(
        _NC, _ROWS, 128)
    b = jnp.pad(post_v, (0, _PAD_N - _POST_N)).reshape(_ROWS, 128)

    def fin(a_ref, b_ref, o_ref):
        g = a_ref[0] + a_ref[1]
        o_ref[...] = (_G_BAR * g) * (_E_AMPA - b_ref[...])

    out = pl.pallas_call(
        fin,
        out_shape=jax.ShapeDtypeStruct((_ROWS, 128), jnp.float32),
    )(a, b)
    return out.reshape(-1)[:_POST_N]


def kernel(pre_spikes, post_v, _row, _col, _val, s):
    col2 = _col.astype(jnp.int32).reshape(_NCHUNK, _CH)
    row2 = _row.astype(jnp.int32).reshape(_NCHUNK, _CH)
    val2 = _val.reshape(_NCHUNK, _CH)
    snew = _tc_snew(pre_spikes, s)
    partial = _sc_partial_g(snew, col2, val2, row2)
    return _tc_finish(partial.reshape(_NC, _POST_N), post_v)
